# baseline (speedup 1.0000x reference)
import functools

import jax
import jax.numpy as jnp
from jax.experimental import pallas as pl
from jax.experimental.pallas import tpu as pltpu

MASK_RATIO = 0.6
BM = 1024
NBUF = 4


def _build_mask(noise, k):
    m = noise.shape[1]
    sub = 128
    rows = m // sub
    bits = jax.lax.bitcast_convert_type(noise, jnp.int32).reshape(rows, sub)

    v = jnp.int32(0)
    c_less = jnp.float32(0.0)
    for bit in range(29, -1, -1):
        cand = v + (1 << bit)
        cnt = jnp.sum((bits < cand).astype(jnp.float32))
        take = cnt < k
        v = jnp.where(take, cand, v)
        c_less = jnp.where(take, cnt, c_less)

    eq = (bits == v).astype(jnp.float32)
    i0 = jax.lax.broadcasted_iota(jnp.int32, (sub, sub), 0)
    i1 = jax.lax.broadcasted_iota(jnp.int32, (sub, sub), 1)
    tri_s = (i0 < i1).astype(jnp.float32)
    inner = jax.lax.dot_general(
        eq, tri_s, (((1,), (0,)), ((), ())),
        preferred_element_type=jnp.float32)
    rowtot = jnp.sum(eq, axis=1)[None, :]
    j0 = jax.lax.broadcasted_iota(jnp.int32, (rows, rows), 0)
    j1 = jax.lax.broadcasted_iota(jnp.int32, (rows, rows), 1)
    tri_r = (j0 < j1).astype(jnp.float32)
    rowexcl = jax.lax.dot_general(
        rowtot, tri_r, (((1,), (0,)), ((), ())),
        preferred_element_type=jnp.float32)
    pre = inner + rowexcl.reshape(rows, 1)

    quota = k - c_less
    masked = (bits < v) | ((eq > 0.0) & (pre < quota))
    return masked.astype(jnp.float32).reshape(1, m)


def _manual_kernel(noise_ref, x_hbm, tok_ref, out_hbm, mask_ref,
                   inbuf, outbuf, insem, outsem, *, k, n, blocks_per_row):
    t = pl.program_id(0)

    def in_copy(s, slot):
        return pltpu.make_async_copy(
            x_hbm.at[pl.ds(s * BM, BM), :], inbuf.at[slot], insem.at[slot])

    def out_copy(s, slot):
        return pltpu.make_async_copy(
            outbuf.at[slot], out_hbm.at[pl.ds(s * BM, BM), :], outsem.at[slot])

    @pl.when(t == 0)
    def _():
        for s in range(NBUF - 1):
            in_copy(s, s).start()

    s_next = t + NBUF - 1

    @pl.when(s_next < n)
    def _():
        in_copy(s_next, s_next % NBUF).start()

    @pl.when(t % blocks_per_row == 0)
    def _():
        bi = t // blocks_per_row
        row = noise_ref[pl.ds(bi, 1), :]
        mask_ref[pl.ds(bi, 1), :] = _build_mask(row, k)

    slot = t % NBUF
    in_copy(t, slot).wait()

    @pl.when(t >= NBUF)
    def _():
        out_copy(t - NBUF, slot).wait()

    bi = t // blocks_per_row
    off = (t % blocks_per_row) * BM
    mrow = mask_ref[pl.ds(bi, 1), pl.ds(off, BM)]
    sel = mrow.reshape(BM, 1) > 0.5
    tok = tok_ref[0, 0][None, :]
    outbuf[pl.ds(slot, 1)] = jnp.where(sel, tok, inbuf[pl.ds(slot, 1)])

    out_copy(t, slot).start()

    @pl.when(t == n - 1)
    def _():
        for d in range(NBUF):
            s_done = n - NBUF + d
            out_copy(s_done, s_done % NBUF).wait()


@jax.jit
def kernel(x, mask_token, noise):
    b, m, c = x.shape
    k = int(m * MASK_RATIO)
    n = (b * m) // BM
    blocks_per_row = m // BM
    xf = x.reshape(b * m, c)

    outf, mask_bool = pl.pallas_call(
        functools.partial(_manual_kernel, k=k, n=n,
                          blocks_per_row=blocks_per_row),
        grid=(n,),
        in_specs=[
            pl.BlockSpec((b, m), lambda t: (0, 0)),
            pl.BlockSpec(memory_space=pl.ANY),
            pl.BlockSpec((1, 1, c), lambda t: (0, 0, 0)),
        ],
        out_specs=[
            pl.BlockSpec(memory_space=pl.ANY),
            pl.BlockSpec((b, m), lambda t: (0, 0)),
        ],
        out_shape=[
            jax.ShapeDtypeStruct((b * m, c), x.dtype),
            jax.ShapeDtypeStruct((b, m), jnp.float32),
        ],
        scratch_shapes=[
            pltpu.VMEM((NBUF, BM, c), jnp.float32),
            pltpu.VMEM((NBUF, BM, c), jnp.float32),
            pltpu.SemaphoreType.DMA((NBUF,)),
            pltpu.SemaphoreType.DMA((NBUF,)),
        ],
        compiler_params=pltpu.CompilerParams(
            dimension_semantics=("arbitrary",),
            vmem_limit_bytes=110 * 1024 * 1024,
        ),
    )(noise, xf, mask_token)

    return (outf.reshape(b, m, c), mask_bool)


# speedup vs baseline: 1.0162x; 1.0162x over previous
import functools

import jax
import jax.numpy as jnp
from jax.experimental import pallas as pl
from jax.experimental.pallas import tpu as pltpu

MASK_RATIO = 0.6
BM = 2048
NBUF = 3


def _build_mask(noise, k):
    m = noise.shape[1]
    sub = 128
    rows = m // sub
    bits = jax.lax.bitcast_convert_type(noise, jnp.int32).reshape(rows, sub)

    v = jnp.int32(0)
    c_less = jnp.float32(0.0)
    for bit in range(29, -1, -1):
        cand = v + (1 << bit)
        cnt = jnp.sum((bits < cand).astype(jnp.float32))
        take = cnt < k
        v = jnp.where(take, cand, v)
        c_less = jnp.where(take, cnt, c_less)

    eq = (bits == v).astype(jnp.float32)
    i0 = jax.lax.broadcasted_iota(jnp.int32, (sub, sub), 0)
    i1 = jax.lax.broadcasted_iota(jnp.int32, (sub, sub), 1)
    tri_s = (i0 < i1).astype(jnp.float32)
    inner = jax.lax.dot_general(
        eq, tri_s, (((1,), (0,)), ((), ())),
        preferred_element_type=jnp.float32)
    rowtot = jnp.sum(eq, axis=1)[None, :]
    j0 = jax.lax.broadcasted_iota(jnp.int32, (rows, rows), 0)
    j1 = jax.lax.broadcasted_iota(jnp.int32, (rows, rows), 1)
    tri_r = (j0 < j1).astype(jnp.float32)
    rowexcl = jax.lax.dot_general(
        rowtot, tri_r, (((1,), (0,)), ((), ())),
        preferred_element_type=jnp.float32)
    pre = inner + rowexcl.reshape(rows, 1)

    quota = k - c_less
    masked = (bits < v) | ((eq > 0.0) & (pre < quota))
    return masked.astype(jnp.float32).reshape(1, m)


def _manual_kernel(noise_ref, x_hbm, tok_ref, out_hbm, mask_ref,
                   inbuf, outbuf, insem, outsem, *, k, n, blocks_per_row):
    t = pl.program_id(0)

    h = BM // 2

    def in_half(s, slot, p):
        return pltpu.make_async_copy(
            x_hbm.at[pl.ds(s * BM + p * h, h), :],
            inbuf.at[slot, pl.ds(p * h, h)], insem.at[slot, p])

    def out_half(s, slot, p):
        return pltpu.make_async_copy(
            outbuf.at[slot, pl.ds(p * h, h)],
            out_hbm.at[pl.ds(s * BM + p * h, h), :], outsem.at[slot, p])

    class _Pair:
        def __init__(self, mk):
            self.mk = mk
        def start(self):
            self.mk(0).start()
            self.mk(1).start()
        def wait(self):
            self.mk(0).wait()
            self.mk(1).wait()

    def in_copy(s, slot):
        return _Pair(lambda p: in_half(s, slot, p))

    def out_copy(s, slot):
        return _Pair(lambda p: out_half(s, slot, p))

    @pl.when(t == 0)
    def _():
        for s in range(NBUF - 1):
            in_copy(s, s).start()

    s_next = t + NBUF - 1

    @pl.when(s_next < n)
    def _():
        in_copy(s_next, s_next % NBUF).start()

    @pl.when(t % blocks_per_row == 0)
    def _():
        bi = t // blocks_per_row
        row = noise_ref[pl.ds(bi, 1), :]
        mask_ref[pl.ds(bi, 1), :] = _build_mask(row, k)

    slot = t % NBUF
    in_copy(t, slot).wait()

    @pl.when(t >= NBUF)
    def _():
        out_copy(t - NBUF, slot).wait()

    bi = t // blocks_per_row
    off = (t % blocks_per_row) * BM
    mrow = mask_ref[pl.ds(bi, 1), pl.ds(off, BM)]
    sel = mrow.reshape(BM, 1) > 0.5
    tok = tok_ref[0, 0][None, :]
    outbuf[pl.ds(slot, 1)] = jnp.where(sel, tok, inbuf[pl.ds(slot, 1)])

    out_copy(t, slot).start()

    @pl.when(t == n - 1)
    def _():
        for d in range(NBUF):
            s_done = n - NBUF + d
            out_copy(s_done, s_done % NBUF).wait()


@jax.jit
def kernel(x, mask_token, noise):
    b, m, c = x.shape
    k = int(m * MASK_RATIO)
    n = (b * m) // BM
    blocks_per_row = m // BM
    xf = x.reshape(b * m, c)

    outf, mask_bool = pl.pallas_call(
        functools.partial(_manual_kernel, k=k, n=n,
                          blocks_per_row=blocks_per_row),
        grid=(n,),
        in_specs=[
            pl.BlockSpec((b, m), lambda t: (0, 0)),
            pl.BlockSpec(memory_space=pl.ANY),
            pl.BlockSpec((1, 1, c), lambda t: (0, 0, 0)),
        ],
        out_specs=[
            pl.BlockSpec(memory_space=pl.ANY),
            pl.BlockSpec((b, m), lambda t: (0, 0)),
        ],
        out_shape=[
            jax.ShapeDtypeStruct((b * m, c), x.dtype),
            jax.ShapeDtypeStruct((b, m), jnp.float32),
        ],
        scratch_shapes=[
            pltpu.VMEM((NBUF, BM, c), jnp.float32),
            pltpu.VMEM((NBUF, BM, c), jnp.float32),
            pltpu.SemaphoreType.DMA((NBUF, 2)),
            pltpu.SemaphoreType.DMA((NBUF, 2)),
        ],
        compiler_params=pltpu.CompilerParams(
            dimension_semantics=("arbitrary",),
            vmem_limit_bytes=110 * 1024 * 1024,
        ),
    )(noise, xf, mask_token)

    return (outf.reshape(b, m, c), mask_bool)


# speedup vs baseline: 1.0186x; 1.0024x over previous
import functools

import jax
import jax.numpy as jnp
from jax.experimental import pallas as pl
from jax.experimental.pallas import tpu as pltpu

MASK_RATIO = 0.6
BM = 2048
NBUF = 3


def _build_mask(noise, k):
    m = noise.shape[1]
    sub = 128
    rows = m // sub
    bits = jax.lax.bitcast_convert_type(noise, jnp.int32).reshape(rows, sub)

    v = jnp.int32(0)
    c_less = jnp.float32(0.0)
    for bit in range(29, -1, -1):
        cand = v + (1 << bit)
        cnt = jnp.sum((bits < cand).astype(jnp.float32))
        take = cnt < k
        v = jnp.where(take, cand, v)
        c_less = jnp.where(take, cnt, c_less)

    eq = (bits == v).astype(jnp.float32)
    i0 = jax.lax.broadcasted_iota(jnp.int32, (sub, sub), 0)
    i1 = jax.lax.broadcasted_iota(jnp.int32, (sub, sub), 1)
    tri_s = (i0 < i1).astype(jnp.float32)
    inner = jax.lax.dot_general(
        eq, tri_s, (((1,), (0,)), ((), ())),
        preferred_element_type=jnp.float32)
    rowtot = jnp.sum(eq, axis=1)[None, :]
    j0 = jax.lax.broadcasted_iota(jnp.int32, (rows, rows), 0)
    j1 = jax.lax.broadcasted_iota(jnp.int32, (rows, rows), 1)
    tri_r = (j0 < j1).astype(jnp.float32)
    rowexcl = jax.lax.dot_general(
        rowtot, tri_r, (((1,), (0,)), ((), ())),
        preferred_element_type=jnp.float32)
    pre = inner + rowexcl.reshape(rows, 1)

    quota = k - c_less
    masked = (bits < v) | ((eq > 0.0) & (pre < quota))
    return masked.astype(jnp.float32).reshape(1, m)


def _manual_kernel(noise_ref, x_hbm, tok_ref, out_hbm, mask_ref,
                   inbuf, outbuf, insem, outsem, *, k, n, blocks_per_row):
    t = pl.program_id(0)

    def in_copy(s, slot):
        return pltpu.make_async_copy(
            x_hbm.at[pl.ds(s * BM, BM), :], inbuf.at[slot], insem.at[slot])

    def out_copy(s, slot):
        return pltpu.make_async_copy(
            outbuf.at[slot], out_hbm.at[pl.ds(s * BM, BM), :], outsem.at[slot])

    @pl.when(t == 0)
    def _():
        for s in range(NBUF - 1):
            in_copy(s, s).start()

    s_next = t + NBUF - 1

    @pl.when(s_next < n)
    def _():
        in_copy(s_next, s_next % NBUF).start()

    @pl.when(t % blocks_per_row == 0)
    def _():
        bi = t // blocks_per_row
        row = noise_ref[pl.ds(bi, 1), :]
        mask_ref[pl.ds(bi, 1), :] = _build_mask(row, k)

    slot = t % NBUF
    in_copy(t, slot).wait()

    @pl.when(t >= NBUF)
    def _():
        out_copy(t - NBUF, slot).wait()

    bi = t // blocks_per_row
    off = (t % blocks_per_row) * BM
    mrow = mask_ref[pl.ds(bi, 1), pl.ds(off, BM)]
    sel = mrow.reshape(BM, 1) > 0.5
    tok = tok_ref[0, 0][None, :]
    outbuf[pl.ds(slot, 1)] = jnp.where(sel, tok, inbuf[pl.ds(slot, 1)])

    out_copy(t, slot).start()

    @pl.when(t == n - 1)
    def _():
        for d in range(NBUF):
            s_done = n - NBUF + d
            out_copy(s_done, s_done % NBUF).wait()


@jax.jit
def kernel(x, mask_token, noise):
    b, m, c = x.shape
    k = int(m * MASK_RATIO)
    n = (b * m) // BM
    blocks_per_row = m // BM
    xf = x.reshape(b * m, c)

    outf, mask_bool = pl.pallas_call(
        functools.partial(_manual_kernel, k=k, n=n,
                          blocks_per_row=blocks_per_row),
        grid=(n,),
        in_specs=[
            pl.BlockSpec((b, m), lambda t: (0, 0)),
            pl.BlockSpec(memory_space=pl.ANY),
            pl.BlockSpec((1, 1, c), lambda t: (0, 0, 0)),
        ],
        out_specs=[
            pl.BlockSpec(memory_space=pl.ANY),
            pl.BlockSpec((b, m), lambda t: (0, 0)),
        ],
        out_shape=[
            jax.ShapeDtypeStruct((b * m, c), x.dtype),
            jax.ShapeDtypeStruct((b, m), jnp.float32),
        ],
        scratch_shapes=[
            pltpu.VMEM((NBUF, BM, c), jnp.float32),
            pltpu.VMEM((NBUF, BM, c), jnp.float32),
            pltpu.SemaphoreType.DMA((NBUF,)),
            pltpu.SemaphoreType.DMA((NBUF,)),
        ],
        compiler_params=pltpu.CompilerParams(
            dimension_semantics=("arbitrary",),
            vmem_limit_bytes=110 * 1024 * 1024,
        ),
    )(noise, xf, mask_token)

    return (outf.reshape(b, m, c), mask_bool)
